# baseline (speedup 1.0000x reference)
import functools

import jax
import jax.numpy as jnp
from jax import lax
from jax.experimental import pallas as pl
from jax.experimental.pallas import tpu as pltpu
from jax.experimental.pallas import tpu_sc as plsc

VOCAB = 1000000
DIM = 64
OUT = 4
SEQ = 200
BATCH = 4096
LANES = 16
NC, NS = 2, 16
NW = NC * NS
BPW = BATCH // NW

TC_BLOCK = 10000


def _project_body(t_ref, w_ref, o_ref):
    o_ref[...] = lax.dot_general(
        t_ref[...], w_ref[...],
        (((1,), (1,)), ((), ())),
        preferred_element_type=jnp.float32,
        precision=lax.Precision.HIGHEST,
    ) * (1.0 / SEQ)


def _project(table, w_pad):
    return pl.pallas_call(
        _project_body,
        grid=(VOCAB // TC_BLOCK,),
        in_specs=[
            pl.BlockSpec((TC_BLOCK, DIM), lambda i: (i, 0)),
            pl.BlockSpec((LANES, DIM), lambda i: (0, 0)),
        ],
        out_specs=pl.BlockSpec((TC_BLOCK, LANES), lambda i: (i, 0)),
        out_shape=jax.ShapeDtypeStruct((VOCAB, LANES), jnp.float32),
    )(table, w_pad)


def _gather_sum(text, proj, bias_pad):
    mesh = plsc.VectorSubcoreMesh(core_axis_name="c", subcore_axis_name="s")

    @functools.partial(
        pl.kernel,
        out_type=jax.ShapeDtypeStruct((BATCH, LANES), jnp.float32),
        mesh=mesh,
        scratch_types=[
            pltpu.VMEM((SEQ, BPW), jnp.int32),
            pltpu.VMEM((BPW, LANES), jnp.float32),
            pltpu.VMEM((BPW, LANES), jnp.float32),
            pltpu.VMEM((LANES,), jnp.float32),
        ],
        compiler_params=pltpu.CompilerParams(use_tc_tiling_on_sc=False),
    )
    def k(text_hbm, p_hbm, b_hbm, out_hbm, idx_v, rows_v, acc_v, b_v):
        wid = lax.axis_index("s") * NC + lax.axis_index("c")
        base = wid * BPW
        pltpu.sync_copy(text_hbm.at[:, pl.ds(base, BPW)], idx_v)
        pltpu.sync_copy(b_hbm, b_v)
        bias = b_v[...]

        @pl.loop(0, BPW)
        def _(i):
            acc_v[i, :] = bias

        @pl.loop(0, SEQ)
        def _(s):
            pltpu.sync_copy(p_hbm.at[idx_v.at[s]], rows_v)

            @pl.loop(0, BPW)
            def _(i):
                acc_v[i, :] = acc_v[i, :] + rows_v[i, :]

        pltpu.sync_copy(acc_v, out_hbm.at[pl.ds(base, BPW)])

    return k(text, proj, bias_pad)


def kernel(text, table, W, b):
    w_pad = jnp.zeros((LANES, DIM), jnp.float32).at[:OUT].set(W)
    b_pad = jnp.zeros((LANES,), jnp.float32).at[:OUT].set(b)
    proj = _project(table, w_pad)
    sums = _gather_sum(text, proj, b_pad)
    return sums[:, :OUT]


# speedup vs baseline: 1.4018x; 1.4018x over previous
import functools

import jax
import jax.numpy as jnp
from jax import lax
from jax.experimental import pallas as pl
from jax.experimental.pallas import tpu as pltpu
from jax.experimental.pallas import tpu_sc as plsc

VOCAB = 1000000
DIM = 64
OUT = 4
SEQ = 200
BATCH = 4096
LANES = 16
NC, NS = 2, 16
NW = NC * NS
BPW = BATCH // NW

PACK = 128 // LANES
TC_BLOCK = 1000


def _project_body(t_ref, w_ref, o_ref):
    t = t_ref[...]
    w = w_ref[...].astype(jnp.bfloat16)
    outs = []
    for j in range(PACK):
        tj = t[:, j * DIM:(j + 1) * DIM].astype(jnp.bfloat16)
        outs.append(
            lax.dot_general(tj, w, (((1,), (1,)), ((), ())),
                            preferred_element_type=jnp.float32))
    o_ref[...] = jnp.concatenate(outs, axis=1) * (1.0 / SEQ)


def _project(table8, w_pad):
    rows = VOCAB // PACK
    return pl.pallas_call(
        _project_body,
        grid=(rows // TC_BLOCK,),
        in_specs=[
            pl.BlockSpec((TC_BLOCK, PACK * DIM), lambda i: (i, 0)),
            pl.BlockSpec((LANES, DIM), lambda i: (0, 0)),
        ],
        out_specs=pl.BlockSpec((TC_BLOCK, PACK * LANES), lambda i: (i, 0)),
        out_shape=jax.ShapeDtypeStruct((rows, PACK * LANES), jnp.float32),
    )(table8, w_pad)


def _gather_sum(text, proj, bias_pad):
    mesh = plsc.VectorSubcoreMesh(core_axis_name="c", subcore_axis_name="s")

    @functools.partial(
        pl.kernel,
        out_type=jax.ShapeDtypeStruct((BATCH, LANES), jnp.float32),
        mesh=mesh,
        scratch_types=[
            pltpu.VMEM((SEQ, BPW), jnp.int32),
            pltpu.VMEM((2, BPW, LANES), jnp.float32),
            pltpu.VMEM((BPW, LANES), jnp.float32),
            pltpu.VMEM((LANES,), jnp.float32),
            pltpu.SemaphoreType.DMA,
            pltpu.SemaphoreType.DMA,
        ],
        compiler_params=pltpu.CompilerParams(use_tc_tiling_on_sc=False),
    )
    def k(text_hbm, p_hbm, b_hbm, out_hbm, idx_v, rows_v, acc_v, b_v,
          sem0, sem1):
        wid = lax.axis_index("s") * NC + lax.axis_index("c")
        base = wid * BPW
        pltpu.sync_copy(text_hbm.at[:, pl.ds(base, BPW)], idx_v)
        pltpu.sync_copy(b_hbm, b_v)
        bias = b_v[...]

        @pl.loop(0, BPW)
        def _(i):
            acc_v[i, :] = bias

        pltpu.async_copy(p_hbm.at[idx_v.at[0]], rows_v.at[0], sem0)

        @pl.loop(0, SEQ, step=2)
        def _(s):
            pltpu.async_copy(p_hbm.at[idx_v.at[s + 1]], rows_v.at[1], sem1)
            pltpu.make_async_copy(
                p_hbm.at[idx_v.at[s]], rows_v.at[0], sem0).wait()

            @pl.loop(0, BPW)
            def _(i):
                acc_v[i, :] = acc_v[i, :] + rows_v[0, i, :]

            @pl.when(s + 2 < SEQ)
            def _():
                pltpu.async_copy(
                    p_hbm.at[idx_v.at[s + 2]], rows_v.at[0], sem0)

            pltpu.make_async_copy(
                p_hbm.at[idx_v.at[s + 1]], rows_v.at[1], sem1).wait()

            @pl.loop(0, BPW)
            def _(i):
                acc_v[i, :] = acc_v[i, :] + rows_v[1, i, :]

        pltpu.sync_copy(acc_v, out_hbm.at[pl.ds(base, BPW)])

    return k(text, proj, bias_pad)


def kernel(text, table, W, b):
    w_pad = jnp.zeros((LANES, DIM), jnp.float32).at[:OUT].set(W)
    b_pad = jnp.zeros((LANES,), jnp.float32).at[:OUT].set(b)
    table8 = table.reshape(VOCAB // PACK, PACK * DIM)
    proj = _project(table8, w_pad).reshape(VOCAB, LANES)
    sums = _gather_sum(text, proj, b_pad)
    return sums[:, :OUT]


# speedup vs baseline: 1.4237x; 1.0156x over previous
import functools

import jax
import jax.numpy as jnp
from jax import lax
from jax.experimental import pallas as pl
from jax.experimental.pallas import tpu as pltpu
from jax.experimental.pallas import tpu_sc as plsc

VOCAB = 1000000
DIM = 64
OUT = 4
SEQ = 200
BATCH = 4096
LANES = 16
NC, NS = 2, 16
NW = NC * NS
BPW = BATCH // NW

TC_BLOCK = 4000


def _project_body(t_ref, w_ref, o_ref):
    t = t_ref[...]
    w = w_ref[...].astype(jnp.bfloat16)
    dn = (((1,), (1,)), ((), ()))
    even = lax.dot_general(t[:, :DIM].astype(jnp.bfloat16), w, dn,
                           preferred_element_type=jnp.float32)
    odd = lax.dot_general(t[:, DIM:].astype(jnp.bfloat16), w, dn,
                          preferred_element_type=jnp.float32)
    packed = jnp.concatenate([even, odd], axis=1) * (1.0 / SEQ)
    quarter = TC_BLOCK // 4
    o_ref[...] = jnp.concatenate(
        [packed[q * quarter:(q + 1) * quarter] for q in range(4)], axis=1)


def _project(table2, w_pad):
    lines = VOCAB // 2
    return pl.pallas_call(
        _project_body,
        grid=(lines // TC_BLOCK,),
        in_specs=[
            pl.BlockSpec((TC_BLOCK, 128), lambda i: (i, 0)),
            pl.BlockSpec((LANES, DIM), lambda i: (0, 0)),
        ],
        out_specs=pl.BlockSpec((TC_BLOCK // 4, 128), lambda i: (i, 0)),
        out_shape=jax.ShapeDtypeStruct((lines // 4, 128), jnp.float32),
    )(table2, w_pad)


def _div_exact(x, d):
    q = (x.astype(jnp.float32) * (1.0 / d)).astype(jnp.int32)
    r = x - q * d
    q = q + (r >= d).astype(jnp.int32) - (r < 0).astype(jnp.int32)
    return q


def _permute_body(t_ref, o_ref):
    v = t_ref[...]
    i = _div_exact(v, 8 * TC_BLOCK // 4)
    rem = v - i * (8 * TC_BLOCK // 4)
    p = jnp.bitwise_and(rem, 1)
    u = jnp.right_shift(rem, 1)
    q = _div_exact(u, TC_BLOCK // 4)
    k = u - q * (TC_BLOCK // 4)
    o_ref[...] = i * (8 * TC_BLOCK // 4) + 8 * k + 2 * q + p


def _permute_text(text):
    return pl.pallas_call(
        _permute_body,
        grid=(1,),
        in_specs=[pl.BlockSpec((SEQ, BATCH), lambda i: (0, 0))],
        out_specs=pl.BlockSpec((SEQ, BATCH), lambda i: (0, 0)),
        out_shape=jax.ShapeDtypeStruct((SEQ, BATCH), jnp.int32),
    )(text)


def _gather_sum(text, proj, bias_pad):
    mesh = plsc.VectorSubcoreMesh(core_axis_name="c", subcore_axis_name="s")

    @functools.partial(
        pl.kernel,
        out_type=jax.ShapeDtypeStruct((BATCH, LANES), jnp.float32),
        mesh=mesh,
        scratch_types=[
            pltpu.VMEM((SEQ, BPW), jnp.int32),
            pltpu.VMEM((2, BPW, LANES), jnp.float32),
            pltpu.VMEM((BPW, LANES), jnp.float32),
            pltpu.VMEM((LANES,), jnp.float32),
            pltpu.SemaphoreType.DMA,
            pltpu.SemaphoreType.DMA,
        ],
        compiler_params=pltpu.CompilerParams(use_tc_tiling_on_sc=False),
    )
    def k(text_hbm, p_hbm, b_hbm, out_hbm, idx_v, rows_v, acc_v, b_v,
          sem0, sem1):
        wid = lax.axis_index("s") * NC + lax.axis_index("c")
        base = wid * BPW
        pltpu.sync_copy(text_hbm.at[:, pl.ds(base, BPW)], idx_v)
        pltpu.sync_copy(b_hbm, b_v)
        bias = b_v[...]

        @pl.loop(0, BPW)
        def _(i):
            acc_v[i, :] = bias

        pltpu.async_copy(p_hbm.at[idx_v.at[0]], rows_v.at[0], sem0)

        @pl.loop(0, SEQ, step=2)
        def _(s):
            pltpu.async_copy(p_hbm.at[idx_v.at[s + 1]], rows_v.at[1], sem1)
            pltpu.make_async_copy(
                p_hbm.at[idx_v.at[s]], rows_v.at[0], sem0).wait()

            @pl.loop(0, BPW)
            def _(i):
                acc_v[i, :] = acc_v[i, :] + rows_v[0, i, :]

            @pl.when(s + 2 < SEQ)
            def _():
                pltpu.async_copy(
                    p_hbm.at[idx_v.at[s + 2]], rows_v.at[0], sem0)

            pltpu.make_async_copy(
                p_hbm.at[idx_v.at[s + 1]], rows_v.at[1], sem1).wait()

            @pl.loop(0, BPW)
            def _(i):
                acc_v[i, :] = acc_v[i, :] + rows_v[1, i, :]

        pltpu.sync_copy(acc_v, out_hbm.at[pl.ds(base, BPW)])

    return k(text, proj, bias_pad)


def kernel(text, table, W, b):
    w_pad = jnp.zeros((LANES, DIM), jnp.float32).at[:OUT].set(W)
    b_pad = jnp.zeros((LANES,), jnp.float32).at[:OUT].set(b)
    table2 = table.reshape(VOCAB // 2, 2 * DIM)
    proj = _project(table2, w_pad).reshape(VOCAB, LANES)
    sums = _gather_sum(_permute_text(text), proj, b_pad)
    return sums[:, :OUT]


# speedup vs baseline: 5.0971x; 3.5802x over previous
import functools

import jax
import jax.numpy as jnp
from jax import lax
from jax.experimental import pallas as pl
from jax.experimental.pallas import tpu as pltpu
from jax.experimental.pallas import tpu_sc as plsc

VOCAB = 1000000
DIM = 64
OUT = 4
SEQ = 200
BATCH = 4096
LANES = 16
NC, NS = 2, 16
NW = NC * NS
BPW = BATCH // NW

CHUNK = 20
NCHUNK = SEQ // CHUNK

VBLK = 32768
GRID = -(-VOCAB // VBLK)
VPAD = GRID * VBLK
OBLK = VBLK // 8
OSHIFT = OBLK.bit_length() - 1


def _project_body(tT_ref, w_ref, o_ref):
    tT = tT_ref[...].astype(jnp.bfloat16)
    w = (w_ref[...] * (1.0 / SEQ)).astype(jnp.bfloat16)
    packed = lax.dot_general(
        tT, w, (((0,), (1,)), ((), ())),
        preferred_element_type=jnp.float32)
    for s in range(8):
        o_ref[:, s * LANES:(s + 1) * LANES] = packed[s * OBLK:(s + 1) * OBLK]


def _project(tableT, w_pad):
    return pl.pallas_call(
        _project_body,
        grid=(GRID,),
        in_specs=[
            pl.BlockSpec((DIM, VBLK), lambda i: (0, i)),
            pl.BlockSpec((LANES, DIM), lambda i: (0, 0)),
        ],
        out_specs=pl.BlockSpec((OBLK, 128), lambda i: (i, 0)),
        out_shape=jax.ShapeDtypeStruct((VPAD // 8, 128), jnp.float32),
        compiler_params=pltpu.CompilerParams(
            dimension_semantics=("parallel",),
            fuse_transposed_lhs_in_matmul=True,
        ),
    )(tableT, w_pad)


def _permute_body(t_ref, o_ref):
    v = t_ref[...]
    o_ref[...] = (
        jnp.bitwise_and(v, ~(VBLK - 1))
        | jnp.left_shift(jnp.bitwise_and(v, OBLK - 1), 3)
        | jnp.right_shift(jnp.bitwise_and(v, VBLK - 1), OSHIFT)
    )


def _permute_text(text):
    return pl.pallas_call(
        _permute_body,
        grid=(1,),
        in_specs=[pl.BlockSpec((SEQ, BATCH), lambda i: (0, 0))],
        out_specs=pl.BlockSpec((SEQ, BATCH), lambda i: (0, 0)),
        out_shape=jax.ShapeDtypeStruct((SEQ, BATCH), jnp.int32),
    )(text)


def _gather_sum(text, proj, bias_pad):
    mesh = plsc.VectorSubcoreMesh(core_axis_name="c", subcore_axis_name="s")

    @functools.partial(
        pl.kernel,
        out_type=jax.ShapeDtypeStruct((BATCH, LANES), jnp.float32),
        mesh=mesh,
        scratch_types=[
            pltpu.VMEM((SEQ, BPW), jnp.int32),
            pltpu.VMEM((2, CHUNK, BPW, LANES), jnp.float32),
            pltpu.VMEM((BPW, LANES), jnp.float32),
            pltpu.VMEM((LANES,), jnp.float32),
            pltpu.SemaphoreType.DMA,
            pltpu.SemaphoreType.DMA,
        ],
        compiler_params=pltpu.CompilerParams(use_tc_tiling_on_sc=False),
    )
    def k(text_hbm, p_hbm, b_hbm, out_hbm, idx_v, rows_v, acc_v, b_v,
          sem0, sem1):
        wid = lax.axis_index("s") * NC + lax.axis_index("c")
        base = wid * BPW
        pltpu.sync_copy(text_hbm.at[:, pl.ds(base, BPW)], idx_v)
        pltpu.sync_copy(b_hbm, b_v)
        bias = b_v[...]

        @pl.loop(0, BPW)
        def _(i):
            acc_v[i, :] = bias

        def gather(c, buf, sem):
            for r in range(CHUNK):
                pltpu.async_copy(
                    p_hbm.at[idx_v.at[c * CHUNK + r]],
                    rows_v.at[buf, r], sem)

        def wait(c, buf, sem):
            for r in range(CHUNK):
                pltpu.make_async_copy(
                    p_hbm.at[idx_v.at[c * CHUNK + r]],
                    rows_v.at[buf, r], sem).wait()

        def accumulate(buf):
            @pl.loop(0, BPW)
            def _(i):
                acc = acc_v[i, :]
                for r in range(CHUNK):
                    acc = acc + rows_v[buf, r, i, :]
                acc_v[i, :] = acc

        gather(0, 0, sem0)

        @pl.loop(0, NCHUNK, step=2)
        def _(c):
            gather(c + 1, 1, sem1)
            wait(c, 0, sem0)
            accumulate(0)

            @pl.when(c + 2 < NCHUNK)
            def _():
                gather(c + 2, 0, sem0)

            wait(c + 1, 1, sem1)
            accumulate(1)

        pltpu.sync_copy(acc_v, out_hbm.at[pl.ds(base, BPW)])

    return k(text, proj, bias_pad)


def kernel(text, table, W, b):
    w_pad = jnp.zeros((LANES, DIM), jnp.float32).at[:OUT].set(W)
    b_pad = jnp.zeros((LANES,), jnp.float32).at[:OUT].set(b)
    proj = _project(table.T, w_pad).reshape(VPAD, LANES)
    sums = _gather_sum(_permute_text(text), proj, b_pad)
    return sums[:, :OUT]
